# Initial kernel scaffold; baseline (speedup 1.0000x reference)
import jax
import jax.numpy as jnp
from jax.experimental import pallas as pl


def kernel(hidden_states, W_g, W_gate_up, W_down, Ws_gate_up, Ws_down):
    raise NotImplementedError("write your pallas kernel here")


# speedup vs baseline: 1.1397x; 1.1397x over previous
import functools

import jax
import jax.numpy as jnp
from jax import lax
from jax.experimental import pallas as pl
from jax.experimental.pallas import tpu as pltpu
from jax.experimental.pallas import tpu_sc as plsc

E = 8
K = 2
D = 1024
F = 704
T = 2048

NC = 2
NS = 16
NW = NC * NS
TT = T // NW
TM = 128
RT = T * K + NW * TM // 4
NTILES = RT // TM

_sc_mesh = functools.partial(
    pl.kernel,
    mesh=plsc.VectorSubcoreMesh(core_axis_name="c", subcore_axis_name="s"),
)


def _gate_body(x_ref, wg_ref, ids0_ref, ids1_ref, w0_ref, w1_ref):
    x = x_ref[...]
    wg = wg_ref[...]
    logits = jax.lax.dot_general(
        x, wg, (((1,), (1,)), ((), ())),
        preferred_element_type=jnp.float32)
    lanes = jax.lax.broadcasted_iota(jnp.int32, logits.shape, 1)
    m1 = jnp.max(logits, axis=1, keepdims=True)
    i1 = jnp.min(jnp.where(logits == m1, lanes, E), axis=1, keepdims=True)
    masked = jnp.where(lanes == i1, -jnp.inf, logits)
    m2 = jnp.max(masked, axis=1, keepdims=True)
    i2 = jnp.min(jnp.where(masked == m2, lanes, E), axis=1, keepdims=True)
    w0 = 1.0 / (1.0 + jnp.exp(m2 - m1))
    ids0_ref[...] = jnp.broadcast_to(i1, ids0_ref.shape)
    ids1_ref[...] = jnp.broadcast_to(i2, ids1_ref.shape)
    w0_ref[...] = jnp.broadcast_to(w0, w0_ref.shape)
    w1_ref[...] = jnp.broadcast_to(1.0 - w0, w1_ref.shape)


def _gate(x, W_g):
    bm = 256
    out_shapes = [
        jax.ShapeDtypeStruct((T, 8), jnp.int32),
        jax.ShapeDtypeStruct((T, 8), jnp.int32),
        jax.ShapeDtypeStruct((T, 8), jnp.float32),
        jax.ShapeDtypeStruct((T, 8), jnp.float32),
    ]
    specs = [pl.BlockSpec((bm, 8), lambda i: (i, 0)) for _ in range(4)]
    return pl.pallas_call(
        _gate_body,
        grid=(T // bm,),
        in_specs=[
            pl.BlockSpec((bm, D), lambda i: (i, 0)),
            pl.BlockSpec((E, D), lambda i: (0, 0)),
        ],
        out_specs=specs,
        out_shape=out_shapes,
    )(x, W_g)


def _shared_body(x_ref, wgu_ref, wd_ref, out_ref):
    x = x_ref[...]
    h = jnp.dot(x, wgu_ref[...], preferred_element_type=jnp.float32)
    gate = h[:, :F]
    up = h[:, F:]
    act = gate * jax.nn.sigmoid(gate) * up
    out_ref[...] = jnp.dot(act, wd_ref[...], preferred_element_type=jnp.float32)


def _shared(x, Ws_gate_up, Ws_down):
    bm = 256
    return pl.pallas_call(
        _shared_body,
        grid=(T // bm,),
        in_specs=[
            pl.BlockSpec((bm, D), lambda i: (i, 0)),
            pl.BlockSpec((D, 2 * F), lambda i: (0, 0)),
            pl.BlockSpec((F, D), lambda i: (0, 0)),
        ],
        out_specs=pl.BlockSpec((bm, D), lambda i: (i, 0)),
        out_shape=jax.ShapeDtypeStruct((T, D), jnp.float32),
    )(x, Ws_gate_up, Ws_down)


def _wid():
    return lax.axis_index("s") * NC + lax.axis_index("c")


def _unpack_to_smem(v_ref, n, s_ref, s_off):
    for j in range(n // 16):
        v = v_ref[pl.ds(j * 16, 16)]
        for l in range(16):
            s_ref[s_off + j * 16 + l] = v[l]


def _pack_from_smem(s_ref, s_off, n=16):
    lanes = lax.iota(jnp.int32, 16)
    acc = jnp.zeros((16,), jnp.int32)
    for l in range(n):
        acc = jnp.where(lanes == l, s_ref[s_off + l], acc)
    return acc


def _sc_counts_body(ids0_hbm, ids1_hbm, counts_hbm,
                    ids_v, cnt_v, ids_s, hist_s):
    wid = _wid()
    base = wid * TT
    pltpu.sync_copy(ids0_hbm.at[pl.ds(base, TT)], ids_v.at[pl.ds(0, TT)])
    pltpu.sync_copy(ids1_hbm.at[pl.ds(base, TT)], ids_v.at[pl.ds(TT, TT)])
    _unpack_to_smem(ids_v, 2 * TT, ids_s, 0)
    for e in range(E):
        hist_s[e] = 0

    def _hist(i, _):
        e = ids_s[i]
        hist_s[e] = hist_s[e] + 1
        return 0

    lax.fori_loop(0, 2 * TT, _hist, 0)
    cnt_v[...] = _pack_from_smem(hist_s, 0, E)
    pltpu.sync_copy(cnt_v, counts_hbm.at[pl.ds(wid * 16, 16)])


def _sc_counts(ids0, ids1):
    return _sc_mesh(
        _sc_counts_body,
        out_type=jax.ShapeDtypeStruct((NW * 16,), jnp.int32),
        scratch_types=[
            pltpu.VMEM((2 * TT,), jnp.int32),
            pltpu.VMEM((16,), jnp.int32),
            pltpu.SMEM((2 * TT,), jnp.int32),
            pltpu.SMEM((16,), jnp.int32),
        ],
    )(ids0, ids1)


def _sc_dispatch_body(ids0_hbm, ids1_hbm, counts_hbm, x_hbm,
                      xs_hbm, inv0_hbm, inv1_hbm, og_hbm,
                      ids_v, counts_v, og_v, dest0_v, dest1_v, rows_v,
                      ids_s, dest_s, run_s, off_s):
    wid = _wid()
    base = wid * TT
    pltpu.sync_copy(ids0_hbm.at[pl.ds(base, TT)], ids_v.at[pl.ds(0, TT)])
    pltpu.sync_copy(ids1_hbm.at[pl.ds(base, TT)], ids_v.at[pl.ds(TT, TT)])
    pltpu.sync_copy(counts_hbm, counts_v)
    _unpack_to_smem(ids_v, 2 * TT, ids_s, 0)

    def _acc(r, carry):
        g, pre = carry
        row = counts_v[pl.ds(r * 16, 16)]
        rv = jnp.zeros((16,), jnp.int32) + r
        widv = jnp.zeros((16,), jnp.int32) + wid
        m = lax.shift_right_arithmetic(rv - widv, 31)
        g = g + row
        pre = pre + (row & m)
        return (g, pre)

    g, pre = lax.fori_loop(0, NW, _acc,
                           (jnp.zeros((16,), jnp.int32),
                            jnp.zeros((16,), jnp.int32)))

    carry = jnp.int32(0)
    for e in range(E):
        off_s[e] = carry
        run_s[e] = carry + pre[e]
        ge = g[e]
        carry = carry + (((ge + (TM - 1)) >> 7) << 7)

    def _dest(i, _):
        e = ids_s[i]
        d = run_s[e]
        run_s[e] = d + 1
        dest_s[i] = d
        return 0

    lax.fori_loop(0, 2 * TT, _dest, 0)

    for j in range(TT // 16):
        dest0_v[pl.ds(j * 16, 16)] = _pack_from_smem(dest_s, j * 16)
        dest1_v[pl.ds(j * 16, 16)] = _pack_from_smem(dest_s, TT + j * 16)

    pltpu.sync_copy(dest0_v, inv0_hbm.at[pl.ds(base, TT)])
    pltpu.sync_copy(dest1_v, inv1_hbm.at[pl.ds(base, TT)])

    @pl.when(wid == 0)
    def _publish():
        og_v[...] = _pack_from_smem(off_s, 0, E)
        pltpu.sync_copy(og_v, og_hbm.at[pl.ds(0, 16)])
        og_v[...] = g
        pltpu.sync_copy(og_v, og_hbm.at[pl.ds(16, 16)])

    pltpu.sync_copy(x_hbm.at[pl.ds(base, TT)], rows_v)
    pltpu.sync_copy(rows_v, xs_hbm.at[dest0_v])
    pltpu.sync_copy(rows_v, xs_hbm.at[dest1_v])


def _sc_dispatch(ids0, ids1, counts, x):
    return _sc_mesh(
        _sc_dispatch_body,
        out_type=[
            jax.ShapeDtypeStruct((RT, D), jnp.float32),
            jax.ShapeDtypeStruct((T,), jnp.int32),
            jax.ShapeDtypeStruct((T,), jnp.int32),
            jax.ShapeDtypeStruct((32,), jnp.int32),
        ],
        scratch_types=[
            pltpu.VMEM((2 * TT,), jnp.int32),
            pltpu.VMEM((NW * 16,), jnp.int32),
            pltpu.VMEM((16,), jnp.int32),
            pltpu.VMEM((TT,), jnp.int32),
            pltpu.VMEM((TT,), jnp.int32),
            pltpu.VMEM((TT, D), jnp.float32),
            pltpu.SMEM((2 * TT,), jnp.int32),
            pltpu.SMEM((2 * TT,), jnp.int32),
            pltpu.SMEM((16,), jnp.int32),
            pltpu.SMEM((16,), jnp.int32),
        ],
    )(ids0, ids1, counts, x)


def _expert_of(og_ref, i):
    acc = jnp.int32(0)
    for e in range(E):
        acc += jnp.where(og_ref[e] <= TM * i, 1, 0).astype(jnp.int32)
    return acc - 1


def _gffn_body(og_ref, xs_ref, wgu_ref, wd_ref, y_ref):
    i = pl.program_id(0)
    e = _expert_of(og_ref, i)
    rows = og_ref[e] + og_ref[16 + e] - TM * i

    @pl.when(rows > 0)
    def _compute():
        x = xs_ref[...]
        h = jnp.dot(x, wgu_ref[0], preferred_element_type=jnp.float32)
        gate = h[:, :F]
        up = h[:, F:]
        act = gate * jax.nn.sigmoid(gate) * up
        y_ref[...] = jnp.dot(act, wd_ref[0], preferred_element_type=jnp.float32)


def _grouped_ffn(og, xs, W_gate_up, W_down):
    grid_spec = pltpu.PrefetchScalarGridSpec(
        num_scalar_prefetch=1,
        grid=(NTILES,),
        in_specs=[
            pl.BlockSpec((TM, D), lambda i, og: (i, 0)),
            pl.BlockSpec((1, D, 2 * F), lambda i, og: (_expert_of(og, i), 0, 0)),
            pl.BlockSpec((1, F, D), lambda i, og: (_expert_of(og, i), 0, 0)),
        ],
        out_specs=pl.BlockSpec((TM, D), lambda i, og: (i, 0)),
    )
    return pl.pallas_call(
        _gffn_body,
        grid_spec=grid_spec,
        out_shape=jax.ShapeDtypeStruct((RT, D), jnp.float32),
    )(og, xs, W_gate_up, W_down)


_CH = 16


def _sc_combine_body(y_hbm, sh_hbm, inv0_hbm, inv1_hbm, w0_hbm, w1_hbm,
                     out_hbm, iv0_v, iv1_v, w_v, y0_v, y1_v, sh_v,
                     w_s, sem0, sem1):
    wid = _wid()
    for chunk in range(TT // _CH):
        cb = wid * TT + chunk * _CH
        pltpu.sync_copy(inv0_hbm.at[pl.ds(cb, _CH)], iv0_v)
        pltpu.sync_copy(inv1_hbm.at[pl.ds(cb, _CH)], iv1_v)
        pltpu.sync_copy(w0_hbm.at[pl.ds(cb, _CH)], w_v.at[pl.ds(0, _CH)])
        pltpu.sync_copy(w1_hbm.at[pl.ds(cb, _CH)], w_v.at[pl.ds(_CH, _CH)])
        h0 = pltpu.async_copy(y_hbm.at[iv0_v], y0_v, sem0)
        h1 = pltpu.async_copy(y_hbm.at[iv1_v], y1_v, sem1)
        pltpu.sync_copy(sh_hbm.at[pl.ds(cb, _CH)], sh_v)
        w0b = w_v[pl.ds(0, 16)]
        w1b = w_v[pl.ds(16, 16)]
        for l in range(16):
            w_s[l] = w0b[l]
            w_s[16 + l] = w1b[l]
        h0.wait()
        h1.wait()

        def _token(t, _):
            w0t = w_s[t]
            w1t = w_s[16 + t]
            for j in range(D // 16):
                y0 = y0_v[t, pl.ds(j * 16, 16)]
                y1 = y1_v[t, pl.ds(j * 16, 16)]
                sh = sh_v[t, pl.ds(j * 16, 16)]
                sh_v[t, pl.ds(j * 16, 16)] = sh + w0t * y0 + w1t * y1
            return 0

        lax.fori_loop(0, _CH, _token, 0)
        pltpu.sync_copy(sh_v, out_hbm.at[pl.ds(cb, _CH)])


def _sc_combine(y, shared, inv0, inv1, w0, w1):
    return _sc_mesh(
        _sc_combine_body,
        out_type=jax.ShapeDtypeStruct((T, D), jnp.float32),
        scratch_types=[
            pltpu.VMEM((_CH,), jnp.int32),
            pltpu.VMEM((_CH,), jnp.int32),
            pltpu.VMEM((2 * _CH,), jnp.float32),
            pltpu.VMEM((_CH, D), jnp.float32),
            pltpu.VMEM((_CH, D), jnp.float32),
            pltpu.VMEM((_CH, D), jnp.float32),
            pltpu.SMEM((2 * _CH,), jnp.float32),
            pltpu.SemaphoreType.DMA,
            pltpu.SemaphoreType.DMA,
        ],
    )(y, shared, inv0, inv1, w0, w1)


def kernel(hidden_states, W_g, W_gate_up, W_down, Ws_gate_up, Ws_down):
    b, s, d = hidden_states.shape
    x = hidden_states.reshape(-1, d)
    ids0, ids1, w0, w1 = _gate(x, W_g)
    ids0f, ids1f = ids0[:, 0], ids1[:, 0]
    w0f, w1f = w0[:, 0], w1[:, 0]
    counts = _sc_counts(ids0f, ids1f)
    xs, inv0, inv1, og = _sc_dispatch(ids0f, ids1f, counts, x)
    shared_out = _shared(x, Ws_gate_up, Ws_down)
    y = _grouped_ffn(og, xs, W_gate_up, W_down)
    out = _sc_combine(y, shared_out, inv0, inv1, w0f, w1f)
    return out.reshape(b, s, d)
